# baseline (speedup 1.0000x reference)
import functools

import jax
import jax.numpy as jnp
from jax import lax
from jax.experimental import pallas as pl
from jax.experimental.pallas import tpu as pltpu
from jax.experimental.pallas import tpu_sc as plsc

B = 128
V = 100000
ROWS = 8
NC = 2
NS = 16
NW = NC * NS
RPW = B // NW
NV = V // 16
NBUCKET = 2048
NSUB = 4
CAP = 6144
KMIN = 1000
I32MIN = -2147483648
MASK31 = 0x7FFFFFFF


def _to_key(bits):
    return jnp.where(bits >= 0, bits,
                     jnp.int32(-1) - (bits & jnp.int32(MASK31)))


def _from_key(key):
    bits = jnp.where(key >= 0, key,
                     (jnp.int32(-1) - key) | jnp.int32(I32MIN))
    return lax.bitcast_convert_type(bits, jnp.float32)


U1 = 10
U2 = 4


def _sc_select_body(logits_hbm, scalf_hbm, scali_hbm, mz_hbm,
                    outf_hbm, outi_hbm,
                    row_v, hist_v, ckey_v, cprob_v, cidx_v, sif_v, sii_v,
                    smz_v, sof_v, soi_v):
    wid = lax.axis_index("s") * NC + lax.axis_index("c")
    lane = lax.iota(jnp.int32, 16)
    rbase = wid * RPW

    pltpu.sync_copy(scalf_hbm.at[pl.ds(rbase, RPW)], sif_v)
    pltpu.sync_copy(scali_hbm.at[pl.ds(rbase, RPW)], sii_v)
    pltpu.sync_copy(mz_hbm.at[pl.ds(rbase, RPW)], smz_v)

    for rr in range(RPW):
        row = rbase + rr
        pltpu.sync_copy(logits_hbm.at[row], row_v)
        p = sif_v[rr, pl.ds(0, 16)][0]
        k = sii_v[rr, pl.ds(0, 16)][0]
        mzv = smz_v[rr, pl.ds(0, 16)]
        m = mzv[0]
        z = mzv[1]

        def zero_body(j, _):
            hist_v[pl.ds(j * 16, 16)] = jnp.zeros((16,), jnp.int32)
            return 0
        lax.fori_loop(0, NSUB * NBUCKET // 16, zero_body, 0)

        ones = jnp.ones((16,), jnp.int32)
        sub = (lane & jnp.int32(NSUB - 1)) * jnp.int32(NBUCKET)

        def scan1_body(i, _):
            for u in range(U1):
                v = row_v[pl.ds((i * U1 + u) * 16, 16)]
                bits = lax.bitcast_convert_type(v, jnp.int32)
                key = _to_key(bits)
                bucket = (key >> 21) + jnp.int32(NBUCKET // 2) + sub
                plsc.addupdate_scatter(hist_v, [bucket], ones)
            return 0

        lax.fori_loop(0, NV // U1, scan1_body, 0)

        def tau_body(j, carry):
            acc, best = carry
            j2 = NBUCKET // 16 - 1 - j
            h = hist_v[pl.ds(j2 * 16, 16)]
            for s_ in range(1, NSUB):
                h = h + hist_v[pl.ds(s_ * NBUCKET + j2 * 16, 16)]
            hr = lax.rev(h, (0,))
            cs = plsc.cumsum(hr) + acc
            crossed = cs >= KMIN
            has = jnp.any(crossed)
            ffs = plsc.all_reduce_ffs(crossed)[0]
            bucket_c = j2 * 16 + 15 - ffs
            best = jnp.where((best < 0) & has, bucket_c, best)
            return cs[15], best

        _, best = lax.fori_loop(0, NBUCKET // 16, tau_body,
                                (jnp.int32(0), jnp.int32(-1)))
        tau_key = (best - jnp.int32(NBUCKET // 2)) << 21

        def scan2_body(i, off):
            for u in range(U1):
                iu = i * U1 + u
                v = row_v[pl.ds(iu * 16, 16)]
                bits = lax.bitcast_convert_type(v, jnp.int32)
                key = _to_key(bits)
                msk = (key >= tau_key) & (off <= CAP - 16)
                idx16 = lane + iu * 16
                plsc.store_compressed(ckey_v.at[pl.ds(off, 16)], key,
                                      mask=msk)
                plsc.store_compressed(cidx_v.at[pl.ds(off, 16)], idx16,
                                      mask=msk)
                off = off + plsc.all_reduce_population_count(msk)[0]
            return off

        ncand = lax.fori_loop(0, NV // U1, scan2_body, jnp.int32(0))
        pz = p * z
        km1 = k - jnp.int32(1)
        ng = (ncand + jnp.int32(16 * U2 - 1)) >> (4 + U2.bit_length() - 1)

        def prob_body(j, _):
            for u in range(U2):
                base = (j * U2 + u) * 16
                ki = ckey_v[pl.ds(base, 16)]
                cprob_v[pl.ds(base, 16)] = jnp.exp(_from_key(ki) - m)
            return 0
        lax.fori_loop(0, ng, prob_body, 0)

        def cs_scan(mid):
            def body(j, carry):
                cnt, sv = carry
                for u in range(U2):
                    base = (j * U2 + u) * 16
                    ki = ckey_v[pl.ds(base, 16)]
                    pv = cprob_v[pl.ds(base, 16)]
                    gt = (ki > mid) & ((lane + base) < ncand)
                    cnt = cnt + jnp.where(gt, 1, 0)
                    sv = sv + jnp.where(gt, pv, 0.0)
                return cnt, sv
            cnt, sv = lax.fori_loop(0, ng, body,
                                    (jnp.zeros((16,), jnp.int32),
                                     jnp.zeros((16,), jnp.float32)))
            return jnp.sum(cnt), jnp.sum(sv)

        def bis_cond(state):
            it, lo, hi = state
            return (it < 32) & (hi > lo + 1)

        def bis_body(state):
            it, lo, hi = state
            mid = (lo >> 1) + (hi >> 1) + (lo & hi & jnp.int32(1))
            c, s = cs_scan(mid)
            kp = (c <= km1) & (s <= pz)
            return (it + 1, jnp.where(kp, lo, mid), jnp.where(kp, mid, hi))

        hi0 = _to_key(lax.bitcast_convert_type(m, jnp.int32))
        _, _, ustar = lax.while_loop(bis_cond, bis_body,
                                     (jnp.int32(0), tau_key - jnp.int32(1),
                                      hi0))

        def stats_body(j, carry):
            cnt, sv, ne = carry
            for u in range(U2):
                base = (j * U2 + u) * 16
                ki = ckey_v[pl.ds(base, 16)]
                pv = cprob_v[pl.ds(base, 16)]
                valid = (lane + base) < ncand
                gt = (ki > ustar) & valid
                eq = (ki == ustar) & valid
                cnt = cnt + jnp.where(gt, 1, 0)
                sv = sv + jnp.where(gt, pv, 0.0)
                ne = ne + jnp.where(eq, 1, 0)
            return cnt, sv, ne

        cntv, sgv, nev = lax.fori_loop(
            0, ng, stats_body,
            (jnp.zeros((16,), jnp.int32), jnp.zeros((16,), jnp.float32),
             jnp.zeros((16,), jnp.int32)))
        c_gt = jnp.sum(cntv)
        s_gt = jnp.sum(sgv)
        n_eq = jnp.sum(nev)

        qv = jnp.exp(jnp.broadcast_to(_from_key(ustar) - m, (16,)))
        r_k = k - c_gt
        ratio_v = jnp.minimum((pz - s_gt) / qv, 1e6)
        r_p = ratio_v.astype(jnp.int32)[0] + jnp.int32(1)
        r = jnp.minimum(jnp.minimum(r_k, r_p), n_eq)

        def tie_body(j, carry):
            cnt, istar = carry
            base = j * 16
            ki = ckey_v[pl.ds(base, 16)]
            ix = cidx_v[pl.ds(base, 16)]
            eq = (ki == ustar) & ((lane + base) < ncand)
            eqi = jnp.where(eq, 1, 0)
            ranks = plsc.cumsum(eqi)
            sel = eq & ((cnt + ranks) <= r)
            istar = jnp.maximum(istar, jnp.max(jnp.where(sel, ix, -1)))
            return cnt + ranks[15], istar

        ncv = (ncand + jnp.int32(15)) >> 4
        _, istar = lax.fori_loop(0, ncv, tie_body,
                                 (jnp.int32(0), jnp.int32(-1)))

        inv_dv = 1.0 / (s_gt + r.astype(jnp.float32) * qv)

        sof_v[rr, pl.ds(0, 16)] = jnp.where(lane == 0, m,
                                            jnp.where(lane == 1, inv_dv,
                                                      0.0))
        soi_v[rr, pl.ds(0, 16)] = jnp.where(lane == 0, ustar,
                                            jnp.where(lane == 1, istar, 0))

    pltpu.sync_copy(sof_v, outf_hbm.at[pl.ds(rbase, RPW)])
    pltpu.sync_copy(soi_v, outi_hbm.at[pl.ds(rbase, RPW)])


_sc_select = functools.partial(
    pl.kernel,
    out_type=[
        jax.ShapeDtypeStruct((B, 16), jnp.float32),
        jax.ShapeDtypeStruct((B, 16), jnp.int32),
    ],
    mesh=plsc.VectorSubcoreMesh(core_axis_name="c", subcore_axis_name="s"),
    compiler_params=pltpu.CompilerParams(needs_layout_passes=False),
    scratch_types=[
        pltpu.VMEM((V,), jnp.float32),
        pltpu.VMEM((NSUB * NBUCKET,), jnp.int32),
        pltpu.VMEM((CAP + 64,), jnp.int32),
        pltpu.VMEM((CAP + 64,), jnp.float32),
        pltpu.VMEM((CAP + 64,), jnp.int32),
        pltpu.VMEM((RPW, 16), jnp.float32),
        pltpu.VMEM((RPW, 16), jnp.int32),
        pltpu.VMEM((RPW, 16), jnp.float32),
        pltpu.VMEM((RPW, 16), jnp.float32),
        pltpu.VMEM((RPW, 16), jnp.int32),
    ],
)(_sc_select_body)


def _tc_mz_block(logits_ref, mz_ref):
    x = logits_ref[...]
    m = jnp.max(x, axis=1, keepdims=True)
    z = jnp.sum(jnp.exp(x - m), axis=1, keepdims=True)
    lane16 = lax.broadcasted_iota(jnp.int32, (ROWS, 16), 1)
    mz_ref[...] = jnp.where(lane16 == 0, m, jnp.where(lane16 == 1, z, 0.0))


def _tc_out_block(logits_ref, f_ref, i_ref, out_ref):
    x = logits_ref[...]
    m = f_ref[:, 0:1]
    inv_d = f_ref[:, 1:2]
    ustar = i_ref[:, 0:1]
    istar = i_ref[:, 1:2]
    bits = lax.bitcast_convert_type(x, jnp.int32)
    key = _to_key(bits)
    iota = lax.broadcasted_iota(jnp.int32, (ROWS, V), 1)
    kept = (key > ustar) | ((key == ustar) & (iota <= istar))
    out_ref[...] = jnp.where(kept, jnp.exp(x - m) * inv_d, 0.0)


def kernel(logits, top_ps, top_ks):
    lane = jnp.arange(16)
    scalf = jnp.where(lane[None, :] == 0,
                      top_ps.astype(jnp.float32)[:, None], 0.0)
    scali = jnp.where(lane[None, :] == 0,
                      top_ks.astype(jnp.int32)[:, None], 0)
    mz = pl.pallas_call(
        _tc_mz_block,
        grid=(B // ROWS,),
        in_specs=[pl.BlockSpec((ROWS, V), lambda i: (i, 0))],
        out_specs=pl.BlockSpec((ROWS, 16), lambda i: (i, 0)),
        out_shape=jax.ShapeDtypeStruct((B, 16), jnp.float32),
    )(logits)
    outf, outi = _sc_select(logits, scalf, scali, mz)
    return pl.pallas_call(
        _tc_out_block,
        grid=(B // ROWS,),
        in_specs=[
            pl.BlockSpec((ROWS, V), lambda i: (i, 0)),
            pl.BlockSpec((ROWS, 16), lambda i: (i, 0)),
            pl.BlockSpec((ROWS, 16), lambda i: (i, 0)),
        ],
        out_specs=pl.BlockSpec((ROWS, V), lambda i: (i, 0)),
        out_shape=jax.ShapeDtypeStruct((B, V), jnp.float32),
    )(logits, outf, outi)


# speedup vs baseline: 1.3430x; 1.3430x over previous
import functools

import jax
import jax.numpy as jnp
from jax import lax
from jax.experimental import pallas as pl
from jax.experimental.pallas import tpu as pltpu
from jax.experimental.pallas import tpu_sc as plsc

B = 128
V = 100000
ROWS = 8
NC = 2
NS = 16
NW = NC * NS
RPW = B // NW
NV = V // 16
NBUCKET = 2048
NSUB = 4
CAP = 6144
KMIN = 1000
I32MIN = -2147483648
MASK31 = 0x7FFFFFFF


def _to_key(bits):
    return jnp.where(bits >= 0, bits,
                     jnp.int32(-1) - (bits & jnp.int32(MASK31)))


def _from_key(key):
    bits = jnp.where(key >= 0, key,
                     (jnp.int32(-1) - key) | jnp.int32(I32MIN))
    return lax.bitcast_convert_type(bits, jnp.float32)


U1 = 10
U2 = 4


def _sc_select_body(logits_hbm, scalf_hbm, scali_hbm, mz_hbm,
                    outf_hbm, outi_hbm,
                    row_v, hist_v, ckey_v, cprob_v, cidx_v, sif_v, sii_v,
                    smz_v, sof_v, soi_v):
    wid = lax.axis_index("s") * NC + lax.axis_index("c")
    lane = lax.iota(jnp.int32, 16)
    rbase = wid * RPW

    pltpu.sync_copy(scalf_hbm.at[pl.ds(rbase, RPW)], sif_v)
    pltpu.sync_copy(scali_hbm.at[pl.ds(rbase, RPW)], sii_v)
    pltpu.sync_copy(mz_hbm.at[pl.ds(rbase, RPW)], smz_v)

    for rr in range(RPW):
        row = rbase + rr
        pltpu.sync_copy(logits_hbm.at[row], row_v)
        p = sif_v[rr, pl.ds(0, 16)][0]
        k = sii_v[rr, pl.ds(0, 16)][0]
        mzv = smz_v[rr, pl.ds(0, 16)]
        m = mzv[0]
        z = mzv[1]

        zeros16 = jnp.zeros((16,), jnp.int32)

        @plsc.parallel_loop(0, NSUB * NBUCKET // 16, unroll=8)
        def _zero_loop(j):
            hist_v[pl.ds(j * 16, 16)] = zeros16

        ones = jnp.ones((16,), jnp.int32)
        sub = (lane & jnp.int32(NSUB - 1)) * jnp.int32(NBUCKET)

        @plsc.parallel_loop(0, NV, unroll=U1)
        def _scan1_loop(i):
            v = row_v[pl.ds(i * 16, 16)]
            bits = lax.bitcast_convert_type(v, jnp.int32)
            key = _to_key(bits)
            bucket = (key >> 21) + jnp.int32(NBUCKET // 2) + sub
            plsc.addupdate_scatter(hist_v, [bucket], ones)

        def tau_body(j, carry):
            acc, best = carry
            j2 = NBUCKET // 16 - 1 - j
            h = hist_v[pl.ds(j2 * 16, 16)]
            for s_ in range(1, NSUB):
                h = h + hist_v[pl.ds(s_ * NBUCKET + j2 * 16, 16)]
            hr = lax.rev(h, (0,))
            cs = plsc.cumsum(hr) + acc
            crossed = cs >= KMIN
            has = jnp.any(crossed)
            ffs = plsc.all_reduce_ffs(crossed)[0]
            bucket_c = j2 * 16 + 15 - ffs
            best = jnp.where((best < 0) & has, bucket_c, best)
            return cs[15], best

        _, best = lax.fori_loop(0, NBUCKET // 16, tau_body,
                                (jnp.int32(0), jnp.int32(-1)))
        tau_key = (best - jnp.int32(NBUCKET // 2)) << 21

        @plsc.parallel_loop(0, NV, unroll=U1, carry=jnp.int32(0))
        def ncand(i, off):
            v = row_v[pl.ds(i * 16, 16)]
            bits = lax.bitcast_convert_type(v, jnp.int32)
            key = _to_key(bits)
            msk = (key >= tau_key) & (off <= CAP - 16)
            idx16 = lane + i * 16
            plsc.store_compressed(ckey_v.at[pl.ds(off, 16)], key, mask=msk)
            plsc.store_compressed(cidx_v.at[pl.ds(off, 16)], idx16,
                                  mask=msk)
            return off + plsc.all_reduce_population_count(msk)[0]

        pz = p * z
        km1 = k - jnp.int32(1)
        ng = (ncand + jnp.int32(16 * U2 - 1)) >> (4 + U2.bit_length() - 1)

        @plsc.parallel_loop(0, ng * U2, unroll=U2)
        def _prob_loop(j):
            base = j * 16
            ki = ckey_v[pl.ds(base, 16)]
            cprob_v[pl.ds(base, 16)] = jnp.exp(_from_key(ki) - m)

        def cs_scan(mid):
            def body(j, carry):
                cnt, sv = carry
                for u in range(U2):
                    base = (j * U2 + u) * 16
                    ki = ckey_v[pl.ds(base, 16)]
                    pv = cprob_v[pl.ds(base, 16)]
                    gt = (ki > mid) & ((lane + base) < ncand)
                    cnt = cnt + jnp.where(gt, 1, 0)
                    sv = sv + jnp.where(gt, pv, 0.0)
                return cnt, sv
            cnt, sv = lax.fori_loop(0, ng, body,
                                    (jnp.zeros((16,), jnp.int32),
                                     jnp.zeros((16,), jnp.float32)))
            return jnp.sum(cnt), jnp.sum(sv)

        def bis_cond(state):
            it, lo, hi = state
            return (it < 32) & (hi > lo + 1)

        def bis_body(state):
            it, lo, hi = state
            mid = (lo >> 1) + (hi >> 1) + (lo & hi & jnp.int32(1))
            c, s = cs_scan(mid)
            kp = (c <= km1) & (s <= pz)
            return (it + 1, jnp.where(kp, lo, mid), jnp.where(kp, mid, hi))

        hi0 = _to_key(lax.bitcast_convert_type(m, jnp.int32))
        _, _, ustar = lax.while_loop(bis_cond, bis_body,
                                     (jnp.int32(0), tau_key - jnp.int32(1),
                                      hi0))

        def stats_body(j, carry):
            cnt, sv, ne = carry
            for u in range(U2):
                base = (j * U2 + u) * 16
                ki = ckey_v[pl.ds(base, 16)]
                pv = cprob_v[pl.ds(base, 16)]
                valid = (lane + base) < ncand
                gt = (ki > ustar) & valid
                eq = (ki == ustar) & valid
                cnt = cnt + jnp.where(gt, 1, 0)
                sv = sv + jnp.where(gt, pv, 0.0)
                ne = ne + jnp.where(eq, 1, 0)
            return cnt, sv, ne

        cntv, sgv, nev = lax.fori_loop(
            0, ng, stats_body,
            (jnp.zeros((16,), jnp.int32), jnp.zeros((16,), jnp.float32),
             jnp.zeros((16,), jnp.int32)))
        c_gt = jnp.sum(cntv)
        s_gt = jnp.sum(sgv)
        n_eq = jnp.sum(nev)

        qv = jnp.exp(jnp.broadcast_to(_from_key(ustar) - m, (16,)))
        r_k = k - c_gt
        ratio_v = jnp.minimum((pz - s_gt) / qv, 1e6)
        r_p = ratio_v.astype(jnp.int32)[0] + jnp.int32(1)
        r = jnp.minimum(jnp.minimum(r_k, r_p), n_eq)

        def tie_body(j, carry):
            cnt, istar = carry
            base = j * 16
            ki = ckey_v[pl.ds(base, 16)]
            ix = cidx_v[pl.ds(base, 16)]
            eq = (ki == ustar) & ((lane + base) < ncand)
            eqi = jnp.where(eq, 1, 0)
            ranks = plsc.cumsum(eqi)
            sel = eq & ((cnt + ranks) <= r)
            istar = jnp.maximum(istar, jnp.max(jnp.where(sel, ix, -1)))
            return cnt + ranks[15], istar

        ncv = (ncand + jnp.int32(15)) >> 4
        _, istar = lax.fori_loop(0, ncv, tie_body,
                                 (jnp.int32(0), jnp.int32(-1)))

        inv_dv = 1.0 / (s_gt + r.astype(jnp.float32) * qv)

        sof_v[rr, pl.ds(0, 16)] = jnp.where(lane == 0, m,
                                            jnp.where(lane == 1, inv_dv,
                                                      0.0))
        soi_v[rr, pl.ds(0, 16)] = jnp.where(lane == 0, ustar,
                                            jnp.where(lane == 1, istar, 0))

    pltpu.sync_copy(sof_v, outf_hbm.at[pl.ds(rbase, RPW)])
    pltpu.sync_copy(soi_v, outi_hbm.at[pl.ds(rbase, RPW)])


_sc_select = functools.partial(
    pl.kernel,
    out_type=[
        jax.ShapeDtypeStruct((B, 16), jnp.float32),
        jax.ShapeDtypeStruct((B, 16), jnp.int32),
    ],
    mesh=plsc.VectorSubcoreMesh(core_axis_name="c", subcore_axis_name="s"),
    compiler_params=pltpu.CompilerParams(needs_layout_passes=False),
    scratch_types=[
        pltpu.VMEM((V,), jnp.float32),
        pltpu.VMEM((NSUB * NBUCKET,), jnp.int32),
        pltpu.VMEM((CAP + 64,), jnp.int32),
        pltpu.VMEM((CAP + 64,), jnp.float32),
        pltpu.VMEM((CAP + 64,), jnp.int32),
        pltpu.VMEM((RPW, 16), jnp.float32),
        pltpu.VMEM((RPW, 16), jnp.int32),
        pltpu.VMEM((RPW, 16), jnp.float32),
        pltpu.VMEM((RPW, 16), jnp.float32),
        pltpu.VMEM((RPW, 16), jnp.int32),
    ],
)(_sc_select_body)


def _tc_mz_block(logits_ref, mz_ref):
    x = logits_ref[...]
    m = jnp.max(x, axis=1, keepdims=True)
    z = jnp.sum(jnp.exp(x - m), axis=1, keepdims=True)
    lane16 = lax.broadcasted_iota(jnp.int32, (ROWS, 16), 1)
    mz_ref[...] = jnp.where(lane16 == 0, m, jnp.where(lane16 == 1, z, 0.0))


def _tc_out_block(logits_ref, f_ref, i_ref, out_ref):
    x = logits_ref[...]
    m = f_ref[:, 0:1]
    inv_d = f_ref[:, 1:2]
    ustar = i_ref[:, 0:1]
    istar = i_ref[:, 1:2]
    bits = lax.bitcast_convert_type(x, jnp.int32)
    key = _to_key(bits)
    iota = lax.broadcasted_iota(jnp.int32, (ROWS, V), 1)
    kept = (key > ustar) | ((key == ustar) & (iota <= istar))
    out_ref[...] = jnp.where(kept, jnp.exp(x - m) * inv_d, 0.0)


def kernel(logits, top_ps, top_ks):
    lane = jnp.arange(16)
    scalf = jnp.where(lane[None, :] == 0,
                      top_ps.astype(jnp.float32)[:, None], 0.0)
    scali = jnp.where(lane[None, :] == 0,
                      top_ks.astype(jnp.int32)[:, None], 0)
    mz = pl.pallas_call(
        _tc_mz_block,
        grid=(B // ROWS,),
        in_specs=[pl.BlockSpec((ROWS, V), lambda i: (i, 0))],
        out_specs=pl.BlockSpec((ROWS, 16), lambda i: (i, 0)),
        out_shape=jax.ShapeDtypeStruct((B, 16), jnp.float32),
    )(logits)
    outf, outi = _sc_select(logits, scalf, scali, mz)
    return pl.pallas_call(
        _tc_out_block,
        grid=(B // ROWS,),
        in_specs=[
            pl.BlockSpec((ROWS, V), lambda i: (i, 0)),
            pl.BlockSpec((ROWS, 16), lambda i: (i, 0)),
            pl.BlockSpec((ROWS, 16), lambda i: (i, 0)),
        ],
        out_specs=pl.BlockSpec((ROWS, V), lambda i: (i, 0)),
        out_shape=jax.ShapeDtypeStruct((B, V), jnp.float32),
    )(logits, outf, outi)


# speedup vs baseline: 1.4112x; 1.0508x over previous
import functools

import jax
import jax.numpy as jnp
from jax import lax
from jax.experimental import pallas as pl
from jax.experimental.pallas import tpu as pltpu
from jax.experimental.pallas import tpu_sc as plsc

B = 128
V = 100000
ROWS = 8
NC = 2
NS = 16
NW = NC * NS
RPW = B // NW
NV = V // 16
NBUCKET = 2048
NSUB = 4
CAP = 6144
KMIN = 1000
I32MIN = -2147483648
MASK31 = 0x7FFFFFFF


def _to_key(bits):
    return jnp.where(bits >= 0, bits,
                     jnp.int32(-1) - (bits & jnp.int32(MASK31)))


def _from_key(key):
    bits = jnp.where(key >= 0, key,
                     (jnp.int32(-1) - key) | jnp.int32(I32MIN))
    return lax.bitcast_convert_type(bits, jnp.float32)


U1 = 10
U2 = 4


def _sc_select_body(logits_hbm, scalf_hbm, scali_hbm,
                    outf_hbm, outi_hbm,
                    row_v, hist_v, ckey_v, cprob_v, cidx_v, sif_v, sii_v,
                    sof_v, soi_v):
    wid = lax.axis_index("s") * NC + lax.axis_index("c")
    lane = lax.iota(jnp.int32, 16)
    rbase = wid * RPW

    pltpu.sync_copy(scalf_hbm.at[pl.ds(rbase, RPW)], sif_v)
    pltpu.sync_copy(scali_hbm.at[pl.ds(rbase, RPW)], sii_v)

    for rr in range(RPW):
        row = rbase + rr
        pltpu.sync_copy(logits_hbm.at[row], row_v)
        p = sif_v[rr, pl.ds(0, 16)][0]
        k = sii_v[rr, pl.ds(0, 16)][0]

        zeros16 = jnp.zeros((16,), jnp.int32)

        @plsc.parallel_loop(0, NSUB * NBUCKET // 16, unroll=8)
        def _zero_loop(j):
            hist_v[pl.ds(j * 16, 16)] = zeros16

        ones = jnp.ones((16,), jnp.int32)
        sub = (lane & jnp.int32(NSUB - 1)) * jnp.int32(NBUCKET)

        @plsc.parallel_loop(0, NV, unroll=U1,
                            carry=jnp.full((16,), -jnp.inf, jnp.float32))
        def mv(i, mv_c):
            v = row_v[pl.ds(i * 16, 16)]
            bits = lax.bitcast_convert_type(v, jnp.int32)
            key = _to_key(bits)
            bucket = (key >> 21) + jnp.int32(NBUCKET // 2) + sub
            plsc.addupdate_scatter(hist_v, [bucket], ones)
            return jnp.maximum(mv_c, v)

        m = jnp.max(mv)

        def tau_body(j, carry):
            acc, best = carry
            j2 = NBUCKET // 16 - 1 - j
            h = hist_v[pl.ds(j2 * 16, 16)]
            for s_ in range(1, NSUB):
                h = h + hist_v[pl.ds(s_ * NBUCKET + j2 * 16, 16)]
            hr = lax.rev(h, (0,))
            cs = plsc.cumsum(hr) + acc
            crossed = cs >= KMIN
            has = jnp.any(crossed)
            ffs = plsc.all_reduce_ffs(crossed)[0]
            bucket_c = j2 * 16 + 15 - ffs
            best = jnp.where((best < 0) & has, bucket_c, best)
            return cs[15], best

        _, best = lax.fori_loop(0, NBUCKET // 16, tau_body,
                                (jnp.int32(0), jnp.int32(-1)))
        tau_key = (best - jnp.int32(NBUCKET // 2)) << 21

        @plsc.parallel_loop(0, NV, unroll=U1,
                            carry=(jnp.zeros((16,), jnp.float32),
                                   jnp.int32(0)))
        def zv_ncand(i, carry):
            zv, off = carry
            v = row_v[pl.ds(i * 16, 16)]
            zv = zv + jnp.exp(v - m)
            bits = lax.bitcast_convert_type(v, jnp.int32)
            key = _to_key(bits)
            msk = (key >= tau_key) & (off <= CAP - 16)
            idx16 = lane + i * 16
            plsc.store_compressed(ckey_v.at[pl.ds(off, 16)], key, mask=msk)
            plsc.store_compressed(cidx_v.at[pl.ds(off, 16)], idx16,
                                  mask=msk)
            return zv, off + plsc.all_reduce_population_count(msk)[0]

        zv, ncand = zv_ncand
        z = jnp.sum(zv)
        pz = p * z
        km1 = k - jnp.int32(1)
        ng = (ncand + jnp.int32(16 * U2 - 1)) >> (4 + U2.bit_length() - 1)

        @plsc.parallel_loop(0, ng * U2, unroll=U2)
        def _prob_loop(j):
            base = j * 16
            ki = ckey_v[pl.ds(base, 16)]
            cprob_v[pl.ds(base, 16)] = jnp.exp(_from_key(ki) - m)

        def cs_scan(mid):
            def body(j, carry):
                cnt, sv = carry
                for u in range(U2):
                    base = (j * U2 + u) * 16
                    ki = ckey_v[pl.ds(base, 16)]
                    pv = cprob_v[pl.ds(base, 16)]
                    gt = (ki > mid) & ((lane + base) < ncand)
                    cnt = cnt + jnp.where(gt, 1, 0)
                    sv = sv + jnp.where(gt, pv, 0.0)
                return cnt, sv
            cnt, sv = lax.fori_loop(0, ng, body,
                                    (jnp.zeros((16,), jnp.int32),
                                     jnp.zeros((16,), jnp.float32)))
            return jnp.sum(cnt), jnp.sum(sv)

        def bis_cond(state):
            it, lo, hi = state
            return (it < 32) & (hi > lo + 1)

        def bis_body(state):
            it, lo, hi = state
            mid = (lo >> 1) + (hi >> 1) + (lo & hi & jnp.int32(1))
            c, s = cs_scan(mid)
            kp = (c <= km1) & (s <= pz)
            return (it + 1, jnp.where(kp, lo, mid), jnp.where(kp, mid, hi))

        hi0 = _to_key(lax.bitcast_convert_type(m, jnp.int32))
        _, _, ustar = lax.while_loop(bis_cond, bis_body,
                                     (jnp.int32(0), tau_key - jnp.int32(1),
                                      hi0))

        def stats_body(j, carry):
            cnt, sv, ne = carry
            for u in range(U2):
                base = (j * U2 + u) * 16
                ki = ckey_v[pl.ds(base, 16)]
                pv = cprob_v[pl.ds(base, 16)]
                valid = (lane + base) < ncand
                gt = (ki > ustar) & valid
                eq = (ki == ustar) & valid
                cnt = cnt + jnp.where(gt, 1, 0)
                sv = sv + jnp.where(gt, pv, 0.0)
                ne = ne + jnp.where(eq, 1, 0)
            return cnt, sv, ne

        cntv, sgv, nev = lax.fori_loop(
            0, ng, stats_body,
            (jnp.zeros((16,), jnp.int32), jnp.zeros((16,), jnp.float32),
             jnp.zeros((16,), jnp.int32)))
        c_gt = jnp.sum(cntv)
        s_gt = jnp.sum(sgv)
        n_eq = jnp.sum(nev)

        qv = jnp.exp(jnp.broadcast_to(_from_key(ustar) - m, (16,)))
        r_k = k - c_gt
        ratio_v = jnp.minimum((pz - s_gt) / qv, 1e6)
        r_p = ratio_v.astype(jnp.int32)[0] + jnp.int32(1)
        r = jnp.minimum(jnp.minimum(r_k, r_p), n_eq)

        def tie_body(j, carry):
            cnt, istar = carry
            base = j * 16
            ki = ckey_v[pl.ds(base, 16)]
            ix = cidx_v[pl.ds(base, 16)]
            eq = (ki == ustar) & ((lane + base) < ncand)
            eqi = jnp.where(eq, 1, 0)
            ranks = plsc.cumsum(eqi)
            sel = eq & ((cnt + ranks) <= r)
            istar = jnp.maximum(istar, jnp.max(jnp.where(sel, ix, -1)))
            return cnt + ranks[15], istar

        ncv = (ncand + jnp.int32(15)) >> 4
        _, istar = lax.fori_loop(0, ncv, tie_body,
                                 (jnp.int32(0), jnp.int32(-1)))

        inv_dv = 1.0 / (s_gt + r.astype(jnp.float32) * qv)

        sof_v[rr, pl.ds(0, 16)] = jnp.where(lane == 0, m,
                                            jnp.where(lane == 1, inv_dv,
                                                      0.0))
        soi_v[rr, pl.ds(0, 16)] = jnp.where(lane == 0, ustar,
                                            jnp.where(lane == 1, istar, 0))

    pltpu.sync_copy(sof_v, outf_hbm.at[pl.ds(rbase, RPW)])
    pltpu.sync_copy(soi_v, outi_hbm.at[pl.ds(rbase, RPW)])


_sc_select = functools.partial(
    pl.kernel,
    out_type=[
        jax.ShapeDtypeStruct((B, 16), jnp.float32),
        jax.ShapeDtypeStruct((B, 16), jnp.int32),
    ],
    mesh=plsc.VectorSubcoreMesh(core_axis_name="c", subcore_axis_name="s"),
    compiler_params=pltpu.CompilerParams(needs_layout_passes=False),
    scratch_types=[
        pltpu.VMEM((V,), jnp.float32),
        pltpu.VMEM((NSUB * NBUCKET,), jnp.int32),
        pltpu.VMEM((CAP + 64,), jnp.int32),
        pltpu.VMEM((CAP + 64,), jnp.float32),
        pltpu.VMEM((CAP + 64,), jnp.int32),
        pltpu.VMEM((RPW, 16), jnp.float32),
        pltpu.VMEM((RPW, 16), jnp.int32),
        pltpu.VMEM((RPW, 16), jnp.float32),
        pltpu.VMEM((RPW, 16), jnp.int32),
    ],
)(_sc_select_body)


def _tc_out_block(logits_ref, f_ref, i_ref, out_ref):
    x = logits_ref[...]
    m = f_ref[:, 0:1]
    inv_d = f_ref[:, 1:2]
    ustar = i_ref[:, 0:1]
    istar = i_ref[:, 1:2]
    bits = lax.bitcast_convert_type(x, jnp.int32)
    key = _to_key(bits)
    iota = lax.broadcasted_iota(jnp.int32, (ROWS, V), 1)
    kept = (key > ustar) | ((key == ustar) & (iota <= istar))
    out_ref[...] = jnp.where(kept, jnp.exp(x - m) * inv_d, 0.0)


def kernel(logits, top_ps, top_ks):
    lane = jnp.arange(16)
    scalf = jnp.where(lane[None, :] == 0,
                      top_ps.astype(jnp.float32)[:, None], 0.0)
    scali = jnp.where(lane[None, :] == 0,
                      top_ks.astype(jnp.int32)[:, None], 0)
    outf, outi = _sc_select(logits, scalf, scali)
    return pl.pallas_call(
        _tc_out_block,
        grid=(B // ROWS,),
        in_specs=[
            pl.BlockSpec((ROWS, V), lambda i: (i, 0)),
            pl.BlockSpec((ROWS, 16), lambda i: (i, 0)),
            pl.BlockSpec((ROWS, 16), lambda i: (i, 0)),
        ],
        out_specs=pl.BlockSpec((ROWS, V), lambda i: (i, 0)),
        out_shape=jax.ShapeDtypeStruct((B, V), jnp.float32),
    )(logits, outf, outi)
